# baseline (speedup 1.0000x reference)
import jax
import jax.numpy as jnp
from jax.experimental import pallas as pl
from jax.experimental.pallas import tpu as pltpu

_TILE_N = 4608


def _fused_heads_body(x_ref, wc_ref, bc_ref, wr_ref, br_ref, wd_ref, bd_ref,
                      cls_ref, reg_ref, dir_ref):
    xb = x_ref[0]
    cls_ref[0] = (
        jnp.dot(wc_ref[...], xb, preferred_element_type=jnp.float32) + bc_ref[...]
    )
    reg_ref[0] = (
        jnp.dot(wr_ref[...], xb, preferred_element_type=jnp.float32) + br_ref[...]
    )
    dir_ref[0] = (
        jnp.dot(wd_ref[...], xb, preferred_element_type=jnp.float32) + bd_ref[...]
    )


def kernel(x, W_cls, b_cls, W_reg, b_reg, W_dir, b_dir):
    B, C, H, W = x.shape
    N = H * W
    O_cls = W_cls.shape[0]
    O_reg = W_reg.shape[0]
    O_dir = W_dir.shape[0]

    xr = x.reshape(B, C, N)
    n_tiles = pl.cdiv(N, _TILE_N)

    def x_map(b, n):
        return (b, 0, n)

    def const_map(b, n):
        return (0, 0)

    def out_map(b, n):
        return (b, 0, n)

    outs = pl.pallas_call(
        _fused_heads_body,
        grid=(B, n_tiles),
        compiler_params=pltpu.CompilerParams(
            allow_input_fusion=[True, False, False, False, False, False, False],
        ),
        in_specs=[
            pl.BlockSpec((1, C, _TILE_N), x_map),
            pl.BlockSpec((O_cls, C), const_map),
            pl.BlockSpec((O_cls, 1), const_map),
            pl.BlockSpec((O_reg, C), const_map),
            pl.BlockSpec((O_reg, 1), const_map),
            pl.BlockSpec((O_dir, C), const_map),
            pl.BlockSpec((O_dir, 1), const_map),
        ],
        out_specs=[
            pl.BlockSpec((1, O_cls, _TILE_N), out_map),
            pl.BlockSpec((1, O_reg, _TILE_N), out_map),
            pl.BlockSpec((1, O_dir, _TILE_N), out_map),
        ],
        out_shape=[
            jax.ShapeDtypeStruct((B, O_cls, N), jnp.float32),
            jax.ShapeDtypeStruct((B, O_reg, N), jnp.float32),
            jax.ShapeDtypeStruct((B, O_dir, N), jnp.float32),
        ],
    )(
        xr,
        W_cls, b_cls.reshape(O_cls, 1),
        W_reg, b_reg.reshape(O_reg, 1),
        W_dir, b_dir.reshape(O_dir, 1),
    )

    cls_score, bbox_pred, dir_cls = outs
    return (
        cls_score.reshape(B, O_cls, H, W),
        bbox_pred.reshape(B, O_reg, H, W),
        dir_cls.reshape(B, O_dir, H, W),
    )
